# baseline (speedup 1.0000x reference)
import functools

import jax
import jax.numpy as jnp
from jax.experimental import pallas as pl
from jax.experimental.pallas import tpu as pltpu


def _gcn_body(x_ref, adj_ref, w1_ref, b1_ref, w2_ref, b2_ref, out_ref,
              s1_ref, s2_ref, *, nb, bi):
    t = pl.program_id(0)

    @pl.when(t == 0)
    def _():
        s1 = jnp.dot(x_ref[...], w1_ref[...],
                     preferred_element_type=jnp.float32)
        s1_ref[...] = s1.astype(jnp.bfloat16)

    @pl.when(t < nb)
    def _():
        h = jnp.dot(adj_ref[...], s1_ref[...],
                    preferred_element_type=jnp.float32) + b1_ref[...]
        h = jnp.maximum(h, 0.0).astype(jnp.bfloat16)
        row = pl.multiple_of(t * bi, bi)
        s2 = jnp.dot(h, w2_ref[...], preferred_element_type=jnp.float32)
        s2_ref[pl.ds(row, bi), :] = s2.astype(jnp.bfloat16)

    @pl.when(t >= nb)
    def _():
        logits = jnp.dot(adj_ref[...], s2_ref[...],
                         preferred_element_type=jnp.float32) + b2_ref[...]
        out_ref[...] = jax.nn.sigmoid(logits)


def kernel(x, adj, W1, b1, W2, b2):
    n, nfeat = x.shape
    nhid = W1.shape[1]
    bi = 256
    nb = n // bi

    xb = x.astype(jnp.bfloat16)
    adjb = adj.astype(jnp.bfloat16)
    w1b = W1.astype(jnp.bfloat16)
    w2b = W2.astype(jnp.bfloat16)
    b1r = b1.reshape(1, nhid)
    b2r = b2.reshape(1, n)

    body = functools.partial(_gcn_body, nb=nb, bi=bi)

    out = pl.pallas_call(
        body,
        grid=(2 * nb,),
        in_specs=[
            pl.BlockSpec((n, nfeat), lambda t: (0, 0)),
            pl.BlockSpec((bi, n), lambda t: (t % nb, 0)),
            pl.BlockSpec((nfeat, nhid), lambda t: (0, 0)),
            pl.BlockSpec((1, nhid), lambda t: (0, 0)),
            pl.BlockSpec((nhid, n), lambda t: (0, 0)),
            pl.BlockSpec((1, n), lambda t: (0, 0)),
        ],
        out_specs=pl.BlockSpec((bi, n), lambda t: (jnp.maximum(t - nb, 0), 0)),
        out_shape=jax.ShapeDtypeStruct((n, n), jnp.float32),
        scratch_shapes=[
            pltpu.VMEM((n, nhid), jnp.bfloat16),
            pltpu.VMEM((n, n), jnp.bfloat16),
        ],
    )(xb, adjb, w1b, b1r, w2b, b2r)
    return out


# speedup vs baseline: 1.2042x; 1.2042x over previous
import functools

import jax
import jax.numpy as jnp
from jax.experimental import pallas as pl
from jax.experimental.pallas import tpu as pltpu


def _gcn_body(x_ref, adj_ref, w1_ref, b1_ref, w2_ref, b2_ref, out_ref,
              s1_ref, s2_ref, adjb_ref, *, nb, bi):
    t = pl.program_id(0)

    @pl.when(t == 0)
    def _():
        s1 = jnp.dot(x_ref[...], w1_ref[...],
                     preferred_element_type=jnp.float32)
        s1_ref[...] = s1.astype(jnp.bfloat16)

    @pl.when(t < nb)
    def _():
        ab = adj_ref[...].astype(jnp.bfloat16)
        row = pl.multiple_of(t * bi, bi)
        adjb_ref[pl.ds(row, bi), :] = ab
        h = jnp.dot(ab, s1_ref[...],
                    preferred_element_type=jnp.float32) + b1_ref[...]
        h = jnp.maximum(h, 0.0).astype(jnp.bfloat16)
        s2 = jnp.dot(h, w2_ref[...], preferred_element_type=jnp.float32)
        s2_ref[pl.ds(row, bi), :] = s2.astype(jnp.bfloat16)

    @pl.when(t >= nb)
    def _():
        row = pl.multiple_of((t - nb) * bi, bi)
        logits = jnp.dot(adjb_ref[pl.ds(row, bi), :], s2_ref[...],
                         preferred_element_type=jnp.float32) + b2_ref[...]
        out_ref[...] = jax.nn.sigmoid(logits)


def kernel(x, adj, W1, b1, W2, b2):
    n, nfeat = x.shape
    nhid = W1.shape[1]
    bi = 256
    nb = n // bi

    xb = x.astype(jnp.bfloat16)
    w1b = W1.astype(jnp.bfloat16)
    w2b = W2.astype(jnp.bfloat16)
    b1r = b1.reshape(1, nhid)
    b2r = b2.reshape(1, n)

    body = functools.partial(_gcn_body, nb=nb, bi=bi)

    out = pl.pallas_call(
        body,
        grid=(2 * nb,),
        in_specs=[
            pl.BlockSpec((n, nfeat), lambda t: (0, 0)),
            pl.BlockSpec((bi, n), lambda t: (jnp.minimum(t, nb - 1), 0)),
            pl.BlockSpec((nfeat, nhid), lambda t: (0, 0)),
            pl.BlockSpec((1, nhid), lambda t: (0, 0)),
            pl.BlockSpec((nhid, n), lambda t: (0, 0)),
            pl.BlockSpec((1, n), lambda t: (0, 0)),
        ],
        out_specs=pl.BlockSpec((bi, n), lambda t: (jnp.maximum(t - nb, 0), 0)),
        out_shape=jax.ShapeDtypeStruct((n, n), jnp.float32),
        scratch_shapes=[
            pltpu.VMEM((n, nhid), jnp.bfloat16),
            pltpu.VMEM((n, n), jnp.bfloat16),
            pltpu.VMEM((n, n), jnp.bfloat16),
        ],
    )(xb, adj, w1b, b1r, w2b, b2r)
    return out


# speedup vs baseline: 1.3709x; 1.1385x over previous
import functools

import jax
import jax.numpy as jnp
from jax.experimental import pallas as pl
from jax.experimental.pallas import tpu as pltpu


def _gcn_body(x_ref, adj_ref, w1_ref, b1_ref, w2_ref, b2_ref, out_ref,
              s1_ref, s2_ref, *, nb, bi):
    t = pl.program_id(0)

    @pl.when(t == 0)
    def _():
        s1_ref[...] = jnp.dot(x_ref[...], w1_ref[...],
                              preferred_element_type=jnp.float32)

    @pl.when(t < nb)
    def _():
        h = jnp.dot(adj_ref[...], s1_ref[...],
                    preferred_element_type=jnp.float32) + b1_ref[...]
        h = jnp.maximum(h, 0.0)
        row = pl.multiple_of(t * bi, bi)
        s2_ref[pl.ds(row, bi), :] = jnp.dot(
            h, w2_ref[...], preferred_element_type=jnp.float32)

    @pl.when(t >= nb)
    def _():
        logits = jnp.dot(adj_ref[...], s2_ref[...],
                         preferred_element_type=jnp.float32) + b2_ref[...]
        out_ref[...] = jax.nn.sigmoid(logits)


def kernel(x, adj, W1, b1, W2, b2):
    n, nfeat = x.shape
    nhid = W1.shape[1]
    bi = 512
    nb = n // bi

    b1r = b1.reshape(1, nhid)
    b2r = b2.reshape(1, n)

    body = functools.partial(_gcn_body, nb=nb, bi=bi)

    out = pl.pallas_call(
        body,
        grid=(2 * nb,),
        in_specs=[
            pl.BlockSpec((n, nfeat), lambda t: (0, 0)),
            pl.BlockSpec((bi, n), lambda t: (t % nb, 0)),
            pl.BlockSpec((nfeat, nhid), lambda t: (0, 0)),
            pl.BlockSpec((1, nhid), lambda t: (0, 0)),
            pl.BlockSpec((nhid, n), lambda t: (0, 0)),
            pl.BlockSpec((1, n), lambda t: (0, 0)),
        ],
        out_specs=pl.BlockSpec((bi, n), lambda t: (jnp.maximum(t - nb, 0), 0)),
        out_shape=jax.ShapeDtypeStruct((n, n), jnp.float32),
        scratch_shapes=[
            pltpu.VMEM((n, nhid), jnp.float32),
            pltpu.VMEM((n, n), jnp.float32),
        ],
    )(x, adj, W1, b1r, W2, b2r)
    return out
